# baseline (speedup 1.0000x reference)
import functools

import jax
import jax.numpy as jnp
from jax import lax
from jax.experimental import pallas as pl
from jax.experimental.pallas import tpu as pltpu
from jax.experimental.pallas import tpu_sc as plsc

N = 10000
E = 320000
HEADS = 8
HID = 16
OUT = 16

NC = 2
NS = 16
NW = NC * NS
EC1 = 64
EC2 = 128


def _cpw(ec):
    cpw = -(-E // (ec * NW)) + 1
    return cpw + cpw % 2
N_PAD = 10240
RPT = N_PAD // NS


def _mm2_kernel(x_ref, wl_ref, wr_ref, ol_ref, or_ref):
    x = x_ref[...]
    ol_ref[...] = jnp.dot(x, wl_ref[...], preferred_element_type=jnp.float32)
    or_ref[...] = jnp.dot(x, wr_ref[...], preferred_element_type=jnp.float32)


def _mm2(x, wl, wr):
    m, k = x.shape
    n = wl.shape[1]
    bm = 2000
    return pl.pallas_call(
        _mm2_kernel,
        grid=(m // bm,),
        in_specs=[
            pl.BlockSpec((bm, k), lambda i: (i, 0)),
            pl.BlockSpec((k, n), lambda i: (0, 0)),
            pl.BlockSpec((k, n), lambda i: (0, 0)),
        ],
        out_specs=[
            pl.BlockSpec((bm, n), lambda i: (i, 0)),
            pl.BlockSpec((bm, n), lambda i: (i, 0)),
        ],
        out_shape=[
            jax.ShapeDtypeStruct((m, n), jnp.float32),
            jax.ShapeDtypeStruct((m, n), jnp.float32),
        ],
    )(x, wl, wr)


def _combine1_kernel(u_ref, d_ref, b_ref, wl_ref, wr_ref, ol_ref, or_ref):
    u = u_ref[0] + u_ref[1]
    d = d_ref[0] + d_ref[1]
    inv = 1.0 / (d + 1e-16)
    h = u * jnp.repeat(inv, HID, axis=1) + b_ref[...]
    h = jnp.where(h > 0, h, jnp.exp(jnp.minimum(h, 0.0)) - 1.0)
    ol_ref[...] = jnp.dot(h, wl_ref[...], preferred_element_type=jnp.float32)
    or_ref[...] = jnp.dot(h, wr_ref[...], preferred_element_type=jnp.float32)


def _combine1(u, d, b, wl, wr):
    bm = 2000
    f = u.shape[2]
    n = wl.shape[1]
    return pl.pallas_call(
        _combine1_kernel,
        grid=(N // bm,),
        in_specs=[
            pl.BlockSpec((2, bm, f), lambda i: (0, i, 0)),
            pl.BlockSpec((2, bm, 8), lambda i: (0, i, 0)),
            pl.BlockSpec((1, f), lambda i: (0, 0)),
            pl.BlockSpec((f, n), lambda i: (0, 0)),
            pl.BlockSpec((f, n), lambda i: (0, 0)),
        ],
        out_specs=[
            pl.BlockSpec((bm, n), lambda i: (i, 0)),
            pl.BlockSpec((bm, n), lambda i: (i, 0)),
        ],
        out_shape=[
            jax.ShapeDtypeStruct((N, n), jnp.float32),
            jax.ShapeDtypeStruct((N, n), jnp.float32),
        ],
    )(u, d, b.reshape(1, f), wl, wr)


def _combine2_kernel(u_ref, d_ref, b_ref, o_ref):
    u = u_ref[0] + u_ref[1]
    d = d_ref[0] + d_ref[1]
    o_ref[...] = u / (d[:, 0:1] + 1e-16) + b_ref[...]


def _combine2(u, d, b):
    bm = 2000
    f = u.shape[2]
    return pl.pallas_call(
        _combine2_kernel,
        grid=(N // bm,),
        in_specs=[
            pl.BlockSpec((2, bm, f), lambda i: (0, i, 0)),
            pl.BlockSpec((2, bm, 8), lambda i: (0, i, 0)),
            pl.BlockSpec((1, f), lambda i: (0, 0)),
        ],
        out_specs=pl.BlockSpec((bm, f), lambda i: (i, 0)),
        out_shape=jax.ShapeDtypeStruct((N, f), jnp.float32),
    )(u, d, b.reshape(1, f))


def _sc_sweep(xl, xr, pidx, att, heads, ec):
    f = heads * 16
    cpw = _cpw(ec)
    epw = cpw * ec
    mesh = plsc.VectorSubcoreMesh(core_axis_name="c", subcore_axis_name="s")
    zu = jnp.zeros((RPT, f), jnp.float32)
    zd = jnp.zeros((RPT, 8), jnp.float32)

    @functools.partial(
        pl.kernel,
        out_type=[jax.ShapeDtypeStruct((NC, N_PAD, f), jnp.float32),
                  jax.ShapeDtypeStruct((NC, N_PAD, 8), jnp.float32)],
        mesh=mesh,
        compiler_params=pltpu.CompilerParams(
            needs_layout_passes=False, use_tc_tiling_on_sc=False),
        scratch_types=[
            pltpu.VMEM_SHARED((N_PAD, f), jnp.float32),
            pltpu.VMEM_SHARED((N_PAD, 8), jnp.float32),
            pltpu.VMEM((3, 2, ec), jnp.int32),
            pltpu.VMEM((2 * ec, f), jnp.float32),
            pltpu.VMEM((2 * ec, f), jnp.float32),
            pltpu.VMEM((2 * ec, 8), jnp.float32),
            pltpu.VMEM((heads * 16, 16), jnp.float32),
        ] + [pltpu.SemaphoreType.DMA] * 9,
    )
    def k(xl_hbm, xr_hbm, p_hbm, att_hbm, zu_hbm, zd_hbm,
          u_out, d_out,
          u_sh, d_sh, idxp, xlv, xrv, exv, attv,
          gl0, gl1, gr0, gr1, su0, su1, sd0, sd1, isem):
        gl = (gl0, gl1)
        gr = (gr0, gr1)
        su = (su0, su1)
        sd = (sd0, sd1)
        c = lax.axis_index("c")
        s = lax.axis_index("s")
        wid = c * NS + s
        row0 = s * RPT

        pltpu.sync_copy(zu_hbm, u_sh.at[pl.ds(row0, RPT)])
        pltpu.sync_copy(zd_hbm, d_sh.at[pl.ds(row0, RPT)])
        pltpu.sync_copy(att_hbm, attv)

        if heads == 1:
            zv = jnp.zeros((16,), jnp.float32)
            for h in range(1, 8):
                @pl.loop(0, 2 * ec // 16)
                def _z(g, h=h):
                    r = lax.iota(jnp.int32, 16) + g * 16
                    plsc.store_scatter(exv, [r, jnp.full((16,), h, jnp.int32)], zv)

        plsc.subcore_barrier()

        prow = wid * cpw
        pltpu.sync_copy(p_hbm.at[prow], idxp.at[0])
        pltpu.async_copy(xl_hbm.at[idxp.at[0, 0]], xlv.at[pl.ds(0, ec)], gl[0])
        pltpu.async_copy(xr_hbm.at[idxp.at[0, 1]], xrv.at[pl.ds(0, ec)], gr[0])
        pltpu.async_copy(p_hbm.at[prow + 1], idxp.at[1], isem)

        @pl.loop(0, cpw // 2)
        def _pair(jj):
            for b in (0, 1):
                j = jj * 2 + b
                nb = 1 - b
                kcur = j % 3
                knxt = (j + 1) % 3
                kprev = (j + 2) % 3

                @pl.when(j > 0)
                def _():
                    pltpu.make_async_copy(
                        xrv.at[pl.ds(nb * ec, ec)],
                        u_sh.at[idxp.at[kprev, 1]], su[nb]).wait()
                    pltpu.make_async_copy(
                        exv.at[pl.ds(nb * ec, ec)],
                        d_sh.at[idxp.at[kprev, 1]], sd[nb]).wait()

                @pl.when(j + 1 < cpw)
                def _():
                    pltpu.make_async_copy(
                        p_hbm.at[prow + j + 1], idxp.at[knxt], isem).wait()
                    pltpu.async_copy(xl_hbm.at[idxp.at[knxt, 0]],
                                     xlv.at[pl.ds(nb * ec, ec)], gl[nb])
                    pltpu.async_copy(xr_hbm.at[idxp.at[knxt, 1]],
                                     xrv.at[pl.ds(nb * ec, ec)], gr[nb])

                pltpu.make_async_copy(xl_hbm.at[idxp.at[kcur, 0]],
                                      xlv.at[pl.ds(b * ec, ec)], gl[b]).wait()
                pltpu.make_async_copy(xr_hbm.at[idxp.at[kcur, 1]],
                                      xrv.at[pl.ds(b * ec, ec)], gr[b]).wait()

                @pl.when(j + 2 < cpw)
                def _():
                    pltpu.async_copy(p_hbm.at[prow + j + 2], idxp.at[kprev], isem)

                off = wid * epw + j * ec
                gpc = ec // 16

                @pl.loop(0, heads * gpc)
                def _hg(t):
                    h = t // gpc
                    g = t - h * gpc
                    hcol = h * 16
                    lane = lax.iota(jnp.int32, 16)
                    rl = lane + g * 16
                    rb = rl + b * ec
                    valid = (off + rl) < E
                    accs = [jnp.zeros((16,), jnp.float32) for _ in range(4)]
                    avals = []
                    for ch0 in range(16):
                        col = (lane ^ ch0) | hcol
                        a = plsc.load_gather(xlv, [rb, col])
                        bb = plsc.load_gather(xrv, [rb, col])
                        sv = a + bb
                        e = jnp.maximum(sv, 0.2 * sv)
                        accs[ch0 % 4] = accs[ch0 % 4] + attv[hcol + ch0] * e
                        avals.append(a)
                    acc = (accs[0] + accs[1]) + (accs[2] + accs[3])
                    ex = jnp.where(valid, jnp.exp(acc), 0.0)
                    plsc.store_scatter(
                        exv, [rb, jnp.full((16,), 1, jnp.int32) * h], ex)
                    for ch0 in range(16):
                        col = (lane ^ ch0) | hcol
                        plsc.store_scatter(xrv, [rb, col], ex * avals[ch0])

                pltpu.async_copy(xrv.at[pl.ds(b * ec, ec)],
                                 u_sh.at[idxp.at[kcur, 1]], su[b], add=True)
                pltpu.async_copy(exv.at[pl.ds(b * ec, ec)],
                                 d_sh.at[idxp.at[kcur, 1]], sd[b], add=True)

        lastk = (cpw - 1) % 3
        pltpu.make_async_copy(xrv.at[pl.ds(ec, ec)],
                              u_sh.at[idxp.at[lastk, 1]], su[1]).wait()
        pltpu.make_async_copy(exv.at[pl.ds(ec, ec)],
                              d_sh.at[idxp.at[lastk, 1]], sd[1]).wait()

        plsc.subcore_barrier()
        pltpu.sync_copy(u_sh.at[pl.ds(row0, RPT)],
                        u_out.at[c, pl.ds(row0, RPT)])
        pltpu.sync_copy(d_sh.at[pl.ds(row0, RPT)],
                        d_out.at[c, pl.ds(row0, RPT)])

    rot = jnp.arange(16)[:, None] ^ jnp.arange(16)[None, :]
    att_exp = att.reshape(heads, 16)[:, rot].reshape(heads * 16, 16)
    return k(xl, xr, pidx, att_exp, zu, zd)


def kernel(x, edge_index, Wl1, Wr1, att1, b1, Wl2, Wr2, att2, b2):
    def pack(ec):
        padn = _cpw(ec) * ec * NW - E
        sp = jnp.pad(edge_index[0], (0, padn)).reshape(-1, ec)
        dp = jnp.pad(edge_index[1], (0, padn)).reshape(-1, ec)
        return jnp.stack([sp, dp], axis=1)

    xl1, xr1 = _mm2(x, Wl1, Wr1)
    u1, d1 = _sc_sweep(xl1, xr1, pack(EC1), att1, HEADS, EC1)
    xl2, xr2 = _combine1(u1, d1, b1, Wl2, Wr2)
    u2, d2 = _sc_sweep(xl2, xr2, pack(EC2), att2, 1, EC2)
    return _combine2(u2, d2, b2)


# speedup vs baseline: 1.0170x; 1.0170x over previous
import functools

import jax
import jax.numpy as jnp
from jax import lax
from jax.experimental import pallas as pl
from jax.experimental.pallas import tpu as pltpu
from jax.experimental.pallas import tpu_sc as plsc

N = 10000
E = 320000
HEADS = 8
HID = 16
OUT = 16

NC = 2
NS = 16
NW = NC * NS
EC1 = 80
EC2 = 128


def _cpw(ec):
    cpw = -(-E // (ec * NW)) + 1
    return cpw + cpw % 2
N_PAD = 10112
RPT = N_PAD // NS


def _mm2_kernel(x_ref, wl_ref, wr_ref, ol_ref, or_ref):
    x = x_ref[...]
    ol_ref[...] = jnp.dot(x, wl_ref[...], preferred_element_type=jnp.float32)
    or_ref[...] = jnp.dot(x, wr_ref[...], preferred_element_type=jnp.float32)


def _mm2(x, wl, wr):
    m, k = x.shape
    n = wl.shape[1]
    bm = 2000
    return pl.pallas_call(
        _mm2_kernel,
        grid=(m // bm,),
        in_specs=[
            pl.BlockSpec((bm, k), lambda i: (i, 0)),
            pl.BlockSpec((k, n), lambda i: (0, 0)),
            pl.BlockSpec((k, n), lambda i: (0, 0)),
        ],
        out_specs=[
            pl.BlockSpec((bm, n), lambda i: (i, 0)),
            pl.BlockSpec((bm, n), lambda i: (i, 0)),
        ],
        out_shape=[
            jax.ShapeDtypeStruct((m, n), jnp.float32),
            jax.ShapeDtypeStruct((m, n), jnp.float32),
        ],
    )(x, wl, wr)


def _combine1_kernel(u_ref, d_ref, b_ref, wl_ref, wr_ref, ol_ref, or_ref):
    u = u_ref[0] + u_ref[1]
    d = d_ref[0] + d_ref[1]
    inv = 1.0 / (d + 1e-16)
    h = u * jnp.repeat(inv, HID, axis=1) + b_ref[...]
    h = jnp.where(h > 0, h, jnp.exp(jnp.minimum(h, 0.0)) - 1.0)
    ol_ref[...] = jnp.dot(h, wl_ref[...], preferred_element_type=jnp.float32)
    or_ref[...] = jnp.dot(h, wr_ref[...], preferred_element_type=jnp.float32)


def _combine1(u, d, b, wl, wr):
    bm = 2000
    f = u.shape[2]
    n = wl.shape[1]
    return pl.pallas_call(
        _combine1_kernel,
        grid=(N // bm,),
        in_specs=[
            pl.BlockSpec((2, bm, f), lambda i: (0, i, 0)),
            pl.BlockSpec((2, bm, 8), lambda i: (0, i, 0)),
            pl.BlockSpec((1, f), lambda i: (0, 0)),
            pl.BlockSpec((f, n), lambda i: (0, 0)),
            pl.BlockSpec((f, n), lambda i: (0, 0)),
        ],
        out_specs=[
            pl.BlockSpec((bm, n), lambda i: (i, 0)),
            pl.BlockSpec((bm, n), lambda i: (i, 0)),
        ],
        out_shape=[
            jax.ShapeDtypeStruct((N, n), jnp.float32),
            jax.ShapeDtypeStruct((N, n), jnp.float32),
        ],
    )(u, d, b.reshape(1, f), wl, wr)


def _combine2_kernel(u_ref, d_ref, b_ref, o_ref):
    u = u_ref[0] + u_ref[1]
    d = d_ref[0] + d_ref[1]
    o_ref[...] = u / (d[:, 0:1] + 1e-16) + b_ref[...]


def _combine2(u, d, b):
    bm = 2000
    f = u.shape[2]
    return pl.pallas_call(
        _combine2_kernel,
        grid=(N // bm,),
        in_specs=[
            pl.BlockSpec((2, bm, f), lambda i: (0, i, 0)),
            pl.BlockSpec((2, bm, 8), lambda i: (0, i, 0)),
            pl.BlockSpec((1, f), lambda i: (0, 0)),
        ],
        out_specs=pl.BlockSpec((bm, f), lambda i: (i, 0)),
        out_shape=jax.ShapeDtypeStruct((N, f), jnp.float32),
    )(u, d, b.reshape(1, f))


def _sc_sweep(xl, xr, pidx, att, heads, ec):
    f = heads * 16
    cpw = _cpw(ec)
    epw = cpw * ec
    mesh = plsc.VectorSubcoreMesh(core_axis_name="c", subcore_axis_name="s")
    zu = jnp.zeros((RPT, f), jnp.float32)
    zd = jnp.zeros((RPT, 8), jnp.float32)

    @functools.partial(
        pl.kernel,
        out_type=[jax.ShapeDtypeStruct((NC, N_PAD, f), jnp.float32),
                  jax.ShapeDtypeStruct((NC, N_PAD, 8), jnp.float32)],
        mesh=mesh,
        compiler_params=pltpu.CompilerParams(
            needs_layout_passes=False, use_tc_tiling_on_sc=False),
        scratch_types=[
            pltpu.VMEM_SHARED((N_PAD, f), jnp.float32),
            pltpu.VMEM_SHARED((N_PAD, 8), jnp.float32),
            pltpu.VMEM((3, 2, ec), jnp.int32),
            pltpu.VMEM((2 * ec, f), jnp.float32),
            pltpu.VMEM((2 * ec, f), jnp.float32),
            pltpu.VMEM((2 * ec, 8), jnp.float32),
            pltpu.VMEM((heads * 16, 16), jnp.float32),
        ] + [pltpu.SemaphoreType.DMA] * 9,
    )
    def k(xl_hbm, xr_hbm, p_hbm, att_hbm, zu_hbm, zd_hbm,
          u_out, d_out,
          u_sh, d_sh, idxp, xlv, xrv, exv, attv,
          gl0, gl1, gr0, gr1, su0, su1, sd0, sd1, isem):
        gl = (gl0, gl1)
        gr = (gr0, gr1)
        su = (su0, su1)
        sd = (sd0, sd1)
        c = lax.axis_index("c")
        s = lax.axis_index("s")
        wid = c * NS + s
        row0 = s * RPT

        pltpu.sync_copy(zu_hbm, u_sh.at[pl.ds(row0, RPT)])
        pltpu.sync_copy(zd_hbm, d_sh.at[pl.ds(row0, RPT)])
        pltpu.sync_copy(att_hbm, attv)

        if heads == 1:
            zv = jnp.zeros((16,), jnp.float32)
            for h in range(1, 8):
                @pl.loop(0, 2 * ec // 16)
                def _z(g, h=h):
                    r = lax.iota(jnp.int32, 16) + g * 16
                    plsc.store_scatter(exv, [r, jnp.full((16,), h, jnp.int32)], zv)

        plsc.subcore_barrier()

        prow = wid * cpw
        pltpu.sync_copy(p_hbm.at[prow], idxp.at[0])
        pltpu.async_copy(xl_hbm.at[idxp.at[0, 0]], xlv.at[pl.ds(0, ec)], gl[0])
        pltpu.async_copy(xr_hbm.at[idxp.at[0, 1]], xrv.at[pl.ds(0, ec)], gr[0])
        pltpu.async_copy(p_hbm.at[prow + 1], idxp.at[1], isem)

        @pl.loop(0, cpw // 2)
        def _pair(jj):
            for b in (0, 1):
                j = jj * 2 + b
                nb = 1 - b
                kcur = j % 3
                knxt = (j + 1) % 3
                kprev = (j + 2) % 3

                @pl.when(j > 0)
                def _():
                    pltpu.make_async_copy(
                        xrv.at[pl.ds(nb * ec, ec)],
                        u_sh.at[idxp.at[kprev, 1]], su[nb]).wait()
                    pltpu.make_async_copy(
                        exv.at[pl.ds(nb * ec, ec)],
                        d_sh.at[idxp.at[kprev, 1]], sd[nb]).wait()

                @pl.when(j + 1 < cpw)
                def _():
                    pltpu.make_async_copy(
                        p_hbm.at[prow + j + 1], idxp.at[knxt], isem).wait()
                    pltpu.async_copy(xl_hbm.at[idxp.at[knxt, 0]],
                                     xlv.at[pl.ds(nb * ec, ec)], gl[nb])
                    pltpu.async_copy(xr_hbm.at[idxp.at[knxt, 1]],
                                     xrv.at[pl.ds(nb * ec, ec)], gr[nb])

                pltpu.make_async_copy(xl_hbm.at[idxp.at[kcur, 0]],
                                      xlv.at[pl.ds(b * ec, ec)], gl[b]).wait()
                pltpu.make_async_copy(xr_hbm.at[idxp.at[kcur, 1]],
                                      xrv.at[pl.ds(b * ec, ec)], gr[b]).wait()

                @pl.when(j + 2 < cpw)
                def _():
                    pltpu.async_copy(p_hbm.at[prow + j + 2], idxp.at[kprev], isem)

                off = wid * epw + j * ec
                gpc = ec // 16

                @pl.loop(0, heads * gpc)
                def _hg(t):
                    h = t // gpc
                    g = t - h * gpc
                    hcol = h * 16
                    lane = lax.iota(jnp.int32, 16)
                    rl = lane + g * 16
                    rb = rl + b * ec
                    valid = (off + rl) < E
                    accs = [jnp.zeros((16,), jnp.float32) for _ in range(4)]
                    avals = []
                    for ch0 in range(16):
                        col = (lane ^ ch0) | hcol
                        a = plsc.load_gather(xlv, [rb, col])
                        bb = plsc.load_gather(xrv, [rb, col])
                        sv = a + bb
                        e = jnp.maximum(sv, 0.2 * sv)
                        accs[ch0 % 4] = accs[ch0 % 4] + attv[hcol + ch0] * e
                        avals.append(a)
                    acc = (accs[0] + accs[1]) + (accs[2] + accs[3])
                    ex = jnp.where(valid, jnp.exp(acc), 0.0)
                    plsc.store_scatter(
                        exv, [rb, jnp.full((16,), 1, jnp.int32) * h], ex)
                    for ch0 in range(16):
                        col = (lane ^ ch0) | hcol
                        plsc.store_scatter(xrv, [rb, col], ex * avals[ch0])

                pltpu.async_copy(xrv.at[pl.ds(b * ec, ec)],
                                 u_sh.at[idxp.at[kcur, 1]], su[b], add=True)
                pltpu.async_copy(exv.at[pl.ds(b * ec, ec)],
                                 d_sh.at[idxp.at[kcur, 1]], sd[b], add=True)

        lastk = (cpw - 1) % 3
        pltpu.make_async_copy(xrv.at[pl.ds(ec, ec)],
                              u_sh.at[idxp.at[lastk, 1]], su[1]).wait()
        pltpu.make_async_copy(exv.at[pl.ds(ec, ec)],
                              d_sh.at[idxp.at[lastk, 1]], sd[1]).wait()

        plsc.subcore_barrier()
        pltpu.sync_copy(u_sh.at[pl.ds(row0, RPT)],
                        u_out.at[c, pl.ds(row0, RPT)])
        pltpu.sync_copy(d_sh.at[pl.ds(row0, RPT)],
                        d_out.at[c, pl.ds(row0, RPT)])

    rot = jnp.arange(16)[:, None] ^ jnp.arange(16)[None, :]
    att_exp = att.reshape(heads, 16)[:, rot].reshape(heads * 16, 16)
    return k(xl, xr, pidx, att_exp, zu, zd)


def kernel(x, edge_index, Wl1, Wr1, att1, b1, Wl2, Wr2, att2, b2):
    def pack(ec):
        padn = _cpw(ec) * ec * NW - E
        sp = jnp.pad(edge_index[0], (0, padn)).reshape(-1, ec)
        dp = jnp.pad(edge_index[1], (0, padn)).reshape(-1, ec)
        return jnp.stack([sp, dp], axis=1)

    xl1, xr1 = _mm2(x, Wl1, Wr1)
    u1, d1 = _sc_sweep(xl1, xr1, pack(EC1), att1, HEADS, EC1)
    xl2, xr2 = _combine1(u1, d1, b1, Wl2, Wr2)
    u2, d2 = _sc_sweep(xl2, xr2, pack(EC2), att2, 1, EC2)
    return _combine2(u2, d2, b2)
